# speedup vs baseline: 2.0210x; 2.0210x over previous
import functools

import jax
import jax.numpy as jnp
from jax import lax
from jax.experimental import pallas as pl
from jax.experimental.pallas import tpu as pltpu
from jax.experimental.pallas import tpu_sc as plsc

_EPS = 1e-12


def _make_sc_gather(vocab, d, n_tokens, chunk):
  info = plsc.get_sparse_core_info()
  nc, ns = info.num_cores, info.num_subcores
  nw = nc * ns
  per_w = n_tokens // nw
  n_chunks = per_w // chunk
  assert per_w % chunk == 0 and n_tokens % nw == 0

  mesh = plsc.VectorSubcoreMesh(core_axis_name="c", subcore_axis_name="s")

  @functools.partial(
      pl.kernel,
      mesh=mesh,
      out_type=jax.ShapeDtypeStruct((n_tokens, d), jnp.float32),
      scratch_types=[
          pltpu.VMEM((chunk,), jnp.int32),
          pltpu.VMEM((chunk, d), jnp.float32),
          pltpu.SemaphoreType.DMA,
      ],
  )
  def gather_kernel(table_hbm, idx_hbm, out_hbm, idx_v, rows_v, sem):
    wid = lax.axis_index("s") * nc + lax.axis_index("c")

    def body(g, carry):
      base = wid * per_w + g * chunk
      pltpu.sync_copy(idx_hbm.at[pl.ds(base, chunk)], idx_v)
      pltpu.async_copy(table_hbm.at[idx_v], rows_v, sem).wait()
      pltpu.sync_copy(rows_v, out_hbm.at[pl.ds(base, chunk)])
      return carry

    lax.fori_loop(0, n_chunks, body, 0)

  return gather_kernel


def _ln_body(we_ref, seg_ref, pos_ref, t0_ref, t1_ref, g_ref, b_ref, out_ref):
  x = we_ref[...]
  seg = seg_ref[0][:, :1]
  dt = t1_ref[...] - t0_ref[...]
  x = x + pos_ref[...] + t0_ref[...] + seg * dt
  mean = jnp.mean(x, axis=1, keepdims=True)
  xc = x - mean
  var = jnp.mean(xc * xc, axis=1, keepdims=True)
  y = xc * lax.rsqrt(var + _EPS)
  out_ref[0] = y * g_ref[...] + b_ref[...]


def kernel(input_ids, segment_ids, word_emb, pos_emb, type_emb, gamma, beta):
  b, l = input_ids.shape
  vocab, d = word_emb.shape
  n = b * l

  idx = input_ids.reshape(n).astype(jnp.int32)
  we = _make_sc_gather(vocab, d, n, 128)(word_emb, idx)

  segb = jnp.broadcast_to(
      segment_ids.astype(jnp.float32).reshape(b, l, 1), (b, l, 128)
  )

  out = pl.pallas_call(
      _ln_body,
      grid=(b,),
      in_specs=[
          pl.BlockSpec((l, d), lambda i: (i, 0)),
          pl.BlockSpec((1, l, 128), lambda i: (i, 0, 0)),
          pl.BlockSpec((l, d), lambda i: (0, 0)),
          pl.BlockSpec((1, d), lambda i: (0, 0)),
          pl.BlockSpec((1, d), lambda i: (0, 0)),
          pl.BlockSpec((1, d), lambda i: (0, 0)),
          pl.BlockSpec((1, d), lambda i: (0, 0)),
      ],
      out_specs=pl.BlockSpec((1, l, d), lambda i: (i, 0, 0)),
      out_shape=jax.ShapeDtypeStruct((b, l, d), jnp.float32),
  )(
      we,
      segb,
      pos_emb,
      type_emb[0].reshape(1, d),
      type_emb[1].reshape(1, d),
      gamma.reshape(1, d),
      beta.reshape(1, d),
  )
  return out
